# Initial kernel scaffold; baseline (speedup 1.0000x reference)
import jax
import jax.numpy as jnp
from jax.experimental import pallas as pl


def kernel(log_seqs, table):
    raise NotImplementedError("write your pallas kernel here")


# speedup vs baseline: 1.2983x; 1.2983x over previous
import functools

import jax
import jax.numpy as jnp
from jax import lax
from jax.experimental import pallas as pl
from jax.experimental.pallas import tpu as pltpu
from jax.experimental.pallas import tpu_sc as plsc

EMB = 1024
BATCH = 1024
SEQ = 50
N = BATCH * SEQ
NW = 32
PER_W = N // NW
CHUNK = 40
NCHUNK = PER_W // CHUNK


def _sc_gather(table, idx):
    mesh = plsc.VectorSubcoreMesh(core_axis_name="c", subcore_axis_name="s")

    @functools.partial(
        pl.kernel,
        mesh=mesh,
        out_type=jax.ShapeDtypeStruct((N, EMB), jnp.float32),
        scratch_types=[
            pltpu.VMEM((NCHUNK, CHUNK), jnp.int32),
            pltpu.VMEM((CHUNK, EMB), jnp.float32),
            pltpu.VMEM((CHUNK, EMB), jnp.float32),
            pltpu.SemaphoreType.DMA,
            pltpu.SemaphoreType.DMA,
            pltpu.SemaphoreType.DMA,
            pltpu.SemaphoreType.DMA,
        ],
    )
    def gather_kernel(table_hbm, idx_hbm, out_hbm,
                      idx_v, buf0, buf1, g0, g1, s0, s1):
        wid = lax.axis_index("s") * 2 + lax.axis_index("c")
        base = wid * PER_W
        pltpu.sync_copy(idx_hbm.at[wid], idx_v)

        bufs = (buf0, buf1)
        gsems = (g0, g1)
        ssems = (s0, s1)

        def start_gather(c, b):
            pltpu.async_copy(table_hbm.at[idx_v.at[c]], bufs[b], gsems[b])

        def wait_gather(c, b):
            pltpu.make_async_copy(
                table_hbm.at[idx_v.at[c]], bufs[b], gsems[b]).wait()

        def start_scatter(c, b):
            pltpu.async_copy(
                bufs[b], out_hbm.at[pl.ds(base + c * CHUNK, CHUNK)], ssems[b])

        def wait_scatter(c, b):
            pltpu.make_async_copy(
                bufs[b], out_hbm.at[pl.ds(base + c * CHUNK, CHUNK)],
                ssems[b]).wait()

        start_gather(0, 0)
        start_gather(1, 1)

        def body(i, carry):
            c = 2 * i
            wait_gather(c, 0)
            start_scatter(c, 0)
            wait_gather(c + 1, 1)
            start_scatter(c + 1, 1)
            wait_scatter(c, 0)
            start_gather(c + 2, 0)
            wait_scatter(c + 1, 1)
            start_gather(c + 3, 1)
            return carry

        lax.fori_loop(0, NCHUNK // 2 - 1, body, 0)

        c = NCHUNK - 2
        wait_gather(c, 0)
        start_scatter(c, 0)
        wait_gather(c + 1, 1)
        start_scatter(c + 1, 1)
        wait_scatter(c, 0)
        wait_scatter(c + 1, 1)

    return gather_kernel(table, idx)


def kernel(log_seqs, table):
    idx = log_seqs.astype(jnp.int32).reshape(NW, NCHUNK, CHUNK)
    out = _sc_gather(table, idx)
    return out.reshape(BATCH, SEQ, EMB)


# speedup vs baseline: 1.3001x; 1.0013x over previous
import functools

import jax
import jax.numpy as jnp
from jax import lax
from jax.experimental import pallas as pl
from jax.experimental.pallas import tpu as pltpu
from jax.experimental.pallas import tpu_sc as plsc

EMB = 1024
BATCH = 1024
SEQ = 50
N = BATCH * SEQ
NW = 32
PER_W = N // NW
CHUNK = 16
NBUF = 5
NCHUNK = PER_W // CHUNK
NGROUP = NCHUNK // NBUF


def _sc_gather(table, idx):
    mesh = plsc.VectorSubcoreMesh(core_axis_name="c", subcore_axis_name="s")

    @functools.partial(
        pl.kernel,
        mesh=mesh,
        out_type=jax.ShapeDtypeStruct((N, EMB), jnp.float32),
        scratch_types=(
            [pltpu.VMEM((NCHUNK, CHUNK), jnp.int32)]
            + [pltpu.VMEM((CHUNK, EMB), jnp.float32) for _ in range(NBUF)]
            + [pltpu.SemaphoreType.DMA for _ in range(2 * NBUF)]
        ),
    )
    def gather_kernel(table_hbm, idx_hbm, out_hbm, idx_v, *rest):
        bufs = rest[:NBUF]
        gsems = rest[NBUF:2 * NBUF]
        ssems = rest[2 * NBUF:]

        wid = lax.axis_index("s") * 2 + lax.axis_index("c")
        base = wid * PER_W
        pltpu.sync_copy(idx_hbm.at[wid], idx_v)

        def start_gather(c, b):
            pltpu.async_copy(table_hbm.at[idx_v.at[c]], bufs[b], gsems[b])

        def wait_gather(c, b):
            pltpu.make_async_copy(
                table_hbm.at[idx_v.at[c]], bufs[b], gsems[b]).wait()

        def start_scatter(c, b):
            pltpu.async_copy(
                bufs[b], out_hbm.at[pl.ds(base + c * CHUNK, CHUNK)], ssems[b])

        def wait_scatter(c, b):
            pltpu.make_async_copy(
                bufs[b], out_hbm.at[pl.ds(base + c * CHUNK, CHUNK)],
                ssems[b]).wait()

        for k in range(NBUF):
            start_gather(k, k)

        def body(i, carry):
            c0 = NBUF * i
            for k in range(NBUF):
                wait_gather(c0 + k, k)
                start_scatter(c0 + k, k)
            for k in range(NBUF):
                wait_scatter(c0 + k, k)
                start_gather(c0 + NBUF + k, k)
            return carry

        lax.fori_loop(0, NGROUP - 1, body, 0)

        c0 = NCHUNK - NBUF
        for k in range(NBUF):
            wait_gather(c0 + k, k)
            start_scatter(c0 + k, k)
        for k in range(NBUF):
            wait_scatter(c0 + k, k)

    return gather_kernel(table, idx)


def kernel(log_seqs, table):
    idx = log_seqs.astype(jnp.int32).reshape(NW, NCHUNK, CHUNK)
    out = _sc_gather(table, idx)
    return out.reshape(BATCH, SEQ, EMB)


# speedup vs baseline: 1.3002x; 1.0001x over previous
import functools

import jax
import jax.numpy as jnp
from jax import lax
from jax.experimental import pallas as pl
from jax.experimental.pallas import tpu as pltpu
from jax.experimental.pallas import tpu_sc as plsc

EMB = 1024
BATCH = 1024
SEQ = 50
N = BATCH * SEQ
NW = 32
PER_W = N // NW
CHUNK = 40
NBUF = 2
NCHUNK = PER_W // CHUNK
NGROUP = NCHUNK // NBUF


def _sc_gather(table, idx):
    mesh = plsc.VectorSubcoreMesh(core_axis_name="c", subcore_axis_name="s")

    @functools.partial(
        pl.kernel,
        mesh=mesh,
        out_type=jax.ShapeDtypeStruct((N, EMB), jnp.float32),
        scratch_types=(
            [pltpu.VMEM((NCHUNK, CHUNK), jnp.int32)]
            + [pltpu.VMEM((CHUNK, EMB), jnp.float32) for _ in range(NBUF)]
            + [pltpu.SemaphoreType.DMA for _ in range(2 * NBUF)]
        ),
    )
    def gather_kernel(table_hbm, idx_hbm, out_hbm, idx_v, *rest):
        bufs = rest[:NBUF]
        gsems = rest[NBUF:2 * NBUF]
        ssems = rest[2 * NBUF:]

        wid = lax.axis_index("s") * 2 + lax.axis_index("c")
        base = wid * PER_W
        pltpu.sync_copy(idx_hbm.at[wid], idx_v)

        def start_gather(c, b):
            pltpu.async_copy(table_hbm.at[idx_v.at[c]], bufs[b], gsems[b])

        def wait_gather(c, b):
            pltpu.make_async_copy(
                table_hbm.at[idx_v.at[c]], bufs[b], gsems[b]).wait()

        def start_scatter(c, b):
            pltpu.async_copy(
                bufs[b], out_hbm.at[pl.ds(base + c * CHUNK, CHUNK)], ssems[b])

        def wait_scatter(c, b):
            pltpu.make_async_copy(
                bufs[b], out_hbm.at[pl.ds(base + c * CHUNK, CHUNK)],
                ssems[b]).wait()

        for k in range(NBUF):
            start_gather(k, k)

        def body(i, carry):
            c0 = NBUF * i
            for k in range(NBUF):
                wait_gather(c0 + k, k)
                start_scatter(c0 + k, k)
            for k in range(NBUF):
                wait_scatter(c0 + k, k)
                start_gather(c0 + NBUF + k, k)
            return carry

        lax.fori_loop(0, NGROUP - 1, body, 0)

        c0 = NCHUNK - NBUF
        for k in range(NBUF):
            wait_gather(c0 + k, k)
            start_scatter(c0 + k, k)
        for k in range(NBUF):
            wait_scatter(c0 + k, k)

    return gather_kernel(table, idx)


def kernel(log_seqs, table):
    idx = log_seqs.astype(jnp.int32).reshape(NW, NCHUNK, CHUNK)
    out = _sc_gather(table, idx)
    return out.reshape(BATCH, SEQ, EMB)


# speedup vs baseline: 1.6779x; 1.2905x over previous
import functools

import jax
import jax.numpy as jnp
from jax import lax
from jax.experimental import pallas as pl
from jax.experimental.pallas import tpu as pltpu
from jax.experimental.pallas import tpu_sc as plsc

EMB = 1024
BATCH = 1024
SEQ = 50
MAIN = 48
TAIL = SEQ - MAIN
NW = 32
BPW = BATCH // NW
GRP = 4
NGRP = BPW // GRP
NBUF = 2


def _sc_gather(table, midx, tidx):
    mesh = plsc.VectorSubcoreMesh(core_axis_name="c", subcore_axis_name="s")

    @functools.partial(
        pl.kernel,
        mesh=mesh,
        out_type=(
            jax.ShapeDtypeStruct((BATCH, SEQ, EMB), jnp.float32),
            jax.ShapeDtypeStruct((BATCH * TAIL, EMB), jnp.float32),
        ),
        scratch_types=(
            [pltpu.VMEM((BPW, MAIN), jnp.int32),
             pltpu.VMEM((NGRP, GRP * TAIL), jnp.int32)]
            + [pltpu.VMEM((MAIN, EMB), jnp.float32) for _ in range(NBUF)]
            + [pltpu.VMEM((GRP * TAIL, EMB), jnp.float32) for _ in range(NBUF)]
            + [pltpu.SemaphoreType.DMA for _ in range(4 * NBUF)]
        ),
    )
    def gather_kernel(table_hbm, midx_hbm, tidx_hbm, out_hbm, tail_hbm,
                      idx_v, tidx_v, *rest):
        bufs = rest[:NBUF]
        tbufs = rest[NBUF:2 * NBUF]
        gsems = rest[2 * NBUF:3 * NBUF]
        ssems = rest[3 * NBUF:4 * NBUF]
        tgsems = rest[4 * NBUF:5 * NBUF]
        tssems = rest[5 * NBUF:]

        wid = lax.axis_index("s") * 2 + lax.axis_index("c")
        base = wid * BPW
        pltpu.sync_copy(midx_hbm.at[wid], idx_v)
        pltpu.sync_copy(tidx_hbm.at[wid], tidx_v)

        def start_gather(c, b):
            pltpu.async_copy(table_hbm.at[idx_v.at[c]], bufs[b], gsems[b])

        def wait_gather(c, b):
            pltpu.make_async_copy(
                table_hbm.at[idx_v.at[c]], bufs[b], gsems[b]).wait()

        def start_scatter(c, b):
            pltpu.async_copy(
                bufs[b], out_hbm.at[base + c, pl.ds(0, MAIN)], ssems[b])

        def wait_scatter(c, b):
            pltpu.make_async_copy(
                bufs[b], out_hbm.at[base + c, pl.ds(0, MAIN)],
                ssems[b]).wait()

        def t_dst(g):
            return tail_hbm.at[pl.ds(TAIL * base + GRP * TAIL * g, GRP * TAIL)]

        def start_tgather(g):
            pltpu.async_copy(
                table_hbm.at[tidx_v.at[g]], tbufs[g % NBUF], tgsems[g % NBUF])

        def wait_tgather(g):
            pltpu.make_async_copy(
                table_hbm.at[tidx_v.at[g]], tbufs[g % NBUF],
                tgsems[g % NBUF]).wait()

        def start_tscatter(g):
            pltpu.async_copy(tbufs[g % NBUF], t_dst(g), tssems[g % NBUF])

        def wait_tscatter(g):
            pltpu.make_async_copy(
                tbufs[g % NBUF], t_dst(g), tssems[g % NBUF]).wait()

        for k in range(NBUF):
            start_gather(k, k)
            start_tgather(k)

        for i in range(BPW // NBUF):
            c0 = NBUF * i
            for k in range(NBUF):
                wait_gather(c0 + k, k)
                start_scatter(c0 + k, k)
            c1 = c0 + NBUF - 1
            if c1 % GRP == GRP - 1:
                g = c1 // GRP
                wait_tgather(g)
                start_tscatter(g)
                if 1 <= g < NGRP - 1:
                    wait_tscatter(g - 1)
                    start_tgather(g + 1)
            for k in range(NBUF):
                wait_scatter(c0 + k, k)
                if c0 + NBUF + k < BPW:
                    start_gather(c0 + NBUF + k, k)

        wait_tscatter(NGRP - 2)
        wait_tscatter(NGRP - 1)

    return gather_kernel(table, midx, tidx)


BB = 128


def _tc_patch(out_sc, tail):
    def body(tail_ref, _, out_ref):
        out_ref[:, :TAIL, :] = tail_ref[...].reshape(BB, TAIL, EMB)
        out_ref[:, TAIL:, :] = jnp.zeros((BB, 8 - TAIL, EMB), jnp.float32)

    return pl.pallas_call(
        body,
        grid=(BATCH // BB,),
        in_specs=[
            pl.BlockSpec((BB * TAIL, EMB), lambda i: (i, 0)),
            pl.BlockSpec((BB, 8, EMB), lambda i: (i, MAIN // 8, 0)),
        ],
        out_specs=pl.BlockSpec((BB, 8, EMB), lambda i: (i, MAIN // 8, 0)),
        out_shape=jax.ShapeDtypeStruct((BATCH, SEQ, EMB), jnp.float32),
        input_output_aliases={1: 0},
    )(tail, out_sc)


def kernel(log_seqs, table):
    idx = log_seqs.astype(jnp.int32).reshape(NW, BPW, SEQ)
    midx = idx[:, :, :MAIN]
    tidx = idx[:, :, MAIN:].reshape(NW, NGRP, GRP * TAIL)
    out_sc, tail = _sc_gather(table, midx, tidx)
    return _tc_patch(out_sc, tail)


# speedup vs baseline: 3.6881x; 2.1980x over previous
import functools

import jax
import jax.numpy as jnp
from jax import lax
from jax.experimental import pallas as pl
from jax.experimental.pallas import tpu as pltpu
from jax.experimental.pallas import tpu_sc as plsc

EMB = 1024
BATCH = 1024
SEQ = 50
N = BATCH * SEQ
NW = 32
PER_W = N // NW
CHUNK = 16
NBUF = 5
NCHUNK = PER_W // CHUNK
NGROUP = NCHUNK // NBUF


def _sc_gather(table, idx):
    mesh = plsc.VectorSubcoreMesh(core_axis_name="c", subcore_axis_name="s")

    @functools.partial(
        pl.kernel,
        mesh=mesh,
        out_type=jax.ShapeDtypeStruct((N, EMB), jnp.float32),
        scratch_types=(
            [pltpu.VMEM((NCHUNK, CHUNK), jnp.int32)]
            + [pltpu.VMEM((CHUNK, EMB), jnp.float32) for _ in range(NBUF)]
            + [pltpu.SemaphoreType.DMA for _ in range(2 * NBUF)]
        ),
    )
    def gather_kernel(table_hbm, idx_hbm, out_hbm, idx_v, *rest):
        bufs = rest[:NBUF]
        gsems = rest[NBUF:2 * NBUF]
        ssems = rest[2 * NBUF:]

        wid = lax.axis_index("s") * 2 + lax.axis_index("c")
        base = wid * PER_W
        pltpu.sync_copy(idx_hbm.at[wid], idx_v)

        def start_gather(c, b):
            pltpu.async_copy(table_hbm.at[idx_v.at[c]], bufs[b], gsems[b])

        def wait_gather(c, b):
            pltpu.make_async_copy(
                table_hbm.at[idx_v.at[c]], bufs[b], gsems[b]).wait()

        def start_scatter(c, b):
            pltpu.async_copy(
                bufs[b], out_hbm.at[pl.ds(base + c * CHUNK, CHUNK)], ssems[b])

        def wait_scatter(c, b):
            pltpu.make_async_copy(
                bufs[b], out_hbm.at[pl.ds(base + c * CHUNK, CHUNK)],
                ssems[b]).wait()

        for k in range(NBUF):
            start_gather(k, k)

        def body(i, carry):
            c0 = NBUF * i
            for k in range(NBUF):
                wait_gather(c0 + k, k)
                start_scatter(c0 + k, k)
            for k in range(NBUF):
                wait_scatter(c0 + k, k)
                start_gather(c0 + NBUF + k, k)
            return carry

        lax.fori_loop(0, NGROUP - 1, body, 0)

        c0 = NCHUNK - NBUF
        for k in range(NBUF):
            wait_gather(c0 + k, k)
            start_scatter(c0 + k, k)
        for k in range(NBUF):
            wait_scatter(c0 + k, k)

    return gather_kernel(table, idx)


def kernel(log_seqs, table):
    idx = log_seqs.astype(jnp.int32).T.reshape(NW, NCHUNK, CHUNK)
    out = _sc_gather(table, idx)
    return out.reshape(SEQ, BATCH, EMB).transpose(1, 0, 2)
